# baseline (speedup 1.0000x reference)
import jax
import jax.numpy as jnp
from jax import lax
from jax.experimental import pallas as pl
from jax.experimental.pallas import tpu as pltpu

N = 32768
C = 81
D = 3

_BI = 16384
_NBLK = N // _BI


def _select_kernel(lab_ref, tab_ref, out_ref):
    lab = lab_ref[...]
    cio = lax.broadcasted_iota(jnp.int32, (C, _BI), 0)
    mask = cio == lab
    for d in range(D):
        picked = jnp.where(mask, tab_ref[d], 0.0)
        out_ref[pl.ds(d, 1), :] = jnp.sum(picked, axis=0, keepdims=True)


def kernel(depth_pred, labels):
    table = depth_pred.transpose(2, 1, 0)
    lab2d = labels.astype(jnp.int32).reshape(1, N)
    out_t = pl.pallas_call(
        _select_kernel,
        grid=(_NBLK,),
        in_specs=[
            pl.BlockSpec((1, _BI), lambda b: (0, b)),
            pl.BlockSpec((D, C, _BI), lambda b: (0, 0, b)),
        ],
        out_specs=pl.BlockSpec((D, _BI), lambda b: (0, b)),
        out_shape=jax.ShapeDtypeStruct((D, N), jnp.float32),
        compiler_params=pltpu.CompilerParams(
            dimension_semantics=("parallel",),
        ),
    )(lab2d, table)
    return out_t.T


# speedup vs baseline: 1.0358x; 1.0358x over previous
import jax
import jax.numpy as jnp
from jax import lax
from jax.experimental import pallas as pl
from jax.experimental.pallas import tpu as pltpu

N = 32768
C = 81
D = 3

_BI = 8192
_NBLK = N // _BI


def _select_kernel(lab_ref, tab_ref, out_ref):
    lab = lab_ref[...]
    cio = lax.broadcasted_iota(jnp.int32, (C, _BI), 0)
    mask = cio == lab
    for d in range(D):
        picked = jnp.where(mask, tab_ref[d], 0.0)
        out_ref[pl.ds(d, 1), :] = jnp.sum(picked, axis=0, keepdims=True)


def kernel(depth_pred, labels):
    table = depth_pred.transpose(2, 1, 0)
    lab2d = labels.astype(jnp.int32).reshape(1, N)
    out_t = pl.pallas_call(
        _select_kernel,
        grid=(_NBLK,),
        in_specs=[
            pl.BlockSpec((1, _BI), lambda b: (0, b)),
            pl.BlockSpec((D, C, _BI), lambda b: (0, 0, b)),
        ],
        out_specs=pl.BlockSpec((D, _BI), lambda b: (0, b)),
        out_shape=jax.ShapeDtypeStruct((D, N), jnp.float32),
        compiler_params=pltpu.CompilerParams(
            dimension_semantics=("parallel",),
        ),
    )(lab2d, table)
    return out_t.T
